# baseline (speedup 1.0000x reference)
import functools

import jax
import jax.numpy as jnp
from jax import lax
from jax.experimental import pallas as pl
from jax.experimental.pallas import tpu as pltpu
from jax.experimental.pallas import tpu_sc as plsc

NC = 2
NS = 16
NW = NC * NS
CH = 128


def _round_up(a, b):
    return (a + b - 1) // b * b


def _sc_degree(dst3, n_deg, n_chunk):
    mesh = plsc.VectorSubcoreMesh(core_axis_name="c", subcore_axis_name="s")
    zt = n_deg // NS
    wave = 8

    @functools.partial(
        pl.kernel,
        out_type=jax.ShapeDtypeStruct((NC * n_deg,), jnp.float32),
        mesh=mesh,
        scratch_types=[
            pltpu.VMEM((n_chunk, CH), jnp.int32),
            pltpu.VMEM((CH,), jnp.float32),
            pltpu.VMEM((zt,), jnp.float32),
            pltpu.VMEM_SHARED((n_deg,), jnp.float32),
            pltpu.SemaphoreType.DMA,
        ],
    )
    def k(dst_h, out_h, dst_v, ones_v, stage_v, deg_sh, dsem):
        c = lax.axis_index("c")
        s = lax.axis_index("s")
        wid = s * NC + c
        pltpu.sync_copy(dst_h.at[wid], dst_v)
        for j in range(CH // 16):
            ones_v[pl.ds(16 * j, 16)] = jnp.ones((16,), jnp.float32)

        def zbody(i, carry):
            stage_v[pl.ds(i * 16, 16)] = jnp.zeros((16,), jnp.float32)
            return carry

        lax.fori_loop(0, zt // 16, zbody, 0)
        pltpu.sync_copy(stage_v, deg_sh.at[pl.ds(s * zt, zt)])
        plsc.subcore_barrier()

        def body(t, carry):
            j0 = t * wave
            for k_ in range(wave):
                pltpu.async_copy(ones_v, deg_sh.at[dst_v.at[j0 + k_]], dsem,
                                 add=True)
            for k_ in range(wave):
                pltpu.make_async_copy(ones_v, deg_sh.at[dst_v.at[0]],
                                      dsem).wait()
            return carry

        lax.fori_loop(0, n_chunk // wave, body, 0)
        plsc.subcore_barrier()
        pltpu.sync_copy(deg_sh.at[pl.ds(s * zt, zt)], stage_v)
        pltpu.sync_copy(stage_v, out_h.at[pl.ds(c * n_deg + s * zt, zt)])

    return k(dst3).reshape(NC, n_deg)


def _sc_aggregate(table, idx4, n, n_acc, tch, nch0):
    d = table.shape[1]
    mesh = plsc.VectorSubcoreMesh(core_axis_name="c", subcore_axis_name="s")
    zt = n_acc // NS
    ni = 4
    nr = 2
    assert nch0 % 4 == 0 and tch % 4 == 0
    assert min(nch0, tch - nch0) >= 12

    @functools.partial(
        pl.kernel,
        out_type=jax.ShapeDtypeStruct((NC, n_acc, d), jnp.float32),
        mesh=mesh,
        scratch_types=[
            pltpu.VMEM((ni, 2, CH), jnp.int32),
            pltpu.VMEM((nr, CH, d), jnp.float32),
            pltpu.VMEM_SHARED((n_acc, d), jnp.float32),
        ]
        + [pltpu.SemaphoreType.DMA] * (ni + 2 * nr),
    )
    def k(tab_h, idx_h, out_h, idx_v, rows_v, acc_sh, *sems):
        isems = sems[:ni]
        gsems = sems[ni:ni + nr]
        ssems = sems[ni + nr:]
        c = lax.axis_index("c")
        s = lax.axis_index("s")
        base = s * tch + jnp.where(c == 0, 0, nch0)
        ll = jnp.where(c == 0, nch0, tch - nch0)

        def istart(j, sl):
            pltpu.async_copy(idx_h.at[base + j], idx_v.at[sl], isems[sl])

        def iwait(sl):
            pltpu.make_async_copy(idx_h.at[0], idx_v.at[sl],
                                  isems[sl]).wait()

        def gstart(sl, rb):
            pltpu.async_copy(tab_h.at[idx_v.at[sl, 0]], rows_v.at[rb],
                             gsems[rb])

        def gwait(rb):
            pltpu.make_async_copy(tab_h.at[idx_v.at[0, 0]], rows_v.at[rb],
                                  gsems[rb]).wait()

        def sstart(sl, rb):
            pltpu.async_copy(rows_v.at[rb], acc_sh.at[idx_v.at[sl, 1]],
                             ssems[rb], add=True)

        def swait(rb):
            pltpu.make_async_copy(rows_v.at[rb], acc_sh.at[idx_v.at[0, 1]],
                                  ssems[rb]).wait()

        def zrow(i, carry):
            for j in range(d // 16):
                rows_v[0, i, pl.ds(j * 16, 16)] = jnp.zeros((16,), jnp.float32)
            return carry

        lax.fori_loop(0, CH, zrow, 0)

        def zcopy(i, carry):
            pltpu.sync_copy(rows_v.at[0], acc_sh.at[pl.ds(s * zt + i * CH, CH)])
            return carry

        lax.fori_loop(0, zt // CH, zcopy, 0)
        plsc.subcore_barrier()

        def step(j, sl, rb, do_ist):
            iwait((sl + 1) % ni)
            swait(1 - rb)
            gstart((sl + 1) % ni, 1 - rb)
            gwait(rb)
            sstart(sl, rb)
            if do_ist:
                istart(j + 3, (sl + 3) % ni)

        istart(0, 0)
        istart(1, 1)
        istart(2, 2)
        iwait(0)
        gstart(0, 0)
        iwait(1)
        gstart(1, 1)
        gwait(0)
        sstart(0, 0)
        istart(3, 3)
        for j in (1, 2, 3):
            step(j, j % ni, j % nr, True)

        def mbody(t, carry):
            j0 = 4 + t * 4
            for k_ in range(4):
                step(j0 + k_, k_ % ni, k_ % nr, True)
            return carry

        lax.fori_loop(0, (ll - 8) // 4, mbody, 0)
        for k_ in range(3):
            step(ll - 4 + k_, k_, k_ % nr, k_ == 0)
        swait(0)
        gwait(1)
        sstart(3, 1)
        swait(1)
        plsc.subcore_barrier()

        def ocopy(i, carry):
            r0 = s * zt + i * CH
            pltpu.sync_copy(acc_sh.at[pl.ds(r0, CH)], rows_v.at[0])
            pltpu.sync_copy(rows_v.at[0], out_h.at[c, pl.ds(r0, CH)])
            return carry

        lax.fori_loop(0, zt // CH, ocopy, 0)

    return k(table, idx4)


def _c1_body(x_ref, dt_ref, xs_ref, disb_ref):
    dt = dt_ref[...]
    deg = dt[:, 0:1] + dt[:, 1:2] + 1.0
    dis = lax.rsqrt(deg)
    disb = jnp.broadcast_to(dis, x_ref.shape)
    disb_ref[...] = disb
    xs_ref[...] = x_ref[...] * disb


def _tc_scale(x, degt):
    n, d = x.shape
    r = 2000
    g = n // r
    return pl.pallas_call(
        _c1_body,
        grid=(g,),
        in_specs=[
            pl.BlockSpec((r, d), lambda i: (i, 0)),
            pl.BlockSpec((r, NC), lambda i: (i, 0)),
        ],
        out_specs=[
            pl.BlockSpec((r, d), lambda i: (i, 0)),
            pl.BlockSpec((r, d), lambda i: (i, 0)),
        ],
        out_shape=[
            jax.ShapeDtypeStruct((n, d), jnp.float32),
            jax.ShapeDtypeStruct((n, d), jnp.float32),
        ],
    )(x, degt)


def _c2_body(z_ref, xs_ref, disb_ref, w1_ref, b1_ref, w2_ref, o_ref):
    disb = disb_ref[...]
    a = disb * (z_ref[0] + z_ref[1] + xs_ref[...])
    h = jnp.dot(a, w1_ref[...], preferred_element_type=jnp.float32,
                precision=lax.Precision.HIGHEST)
    h = jnp.maximum(h + b1_ref[...], 0.0)
    t = jnp.dot(h, w2_ref[...], preferred_element_type=jnp.float32,
                precision=lax.Precision.HIGHEST)
    o_ref[...] = disb * t


def _tc_layer(z, xs, disb, W1, b1, W2):
    n, d = xs.shape
    h = W1.shape[1]
    r = 2000
    g = n // r
    return pl.pallas_call(
        _c2_body,
        grid=(g,),
        in_specs=[
            pl.BlockSpec((NC, r, d), lambda i: (0, i, 0)),
            pl.BlockSpec((r, d), lambda i: (i, 0)),
            pl.BlockSpec((r, d), lambda i: (i, 0)),
            pl.BlockSpec((d, h), lambda i: (0, 0)),
            pl.BlockSpec((1, h), lambda i: (0, 0)),
            pl.BlockSpec((h, d), lambda i: (0, 0)),
        ],
        out_specs=pl.BlockSpec((r, d), lambda i: (i, 0)),
        out_shape=jax.ShapeDtypeStruct((n, d), jnp.float32),
    )(z, xs, disb, W1, b1, W2)


def _c3_body(u_ref, ts_ref, disb_ref, b2_ref, o_ref):
    o_ref[...] = disb_ref[...] * (u_ref[0] + u_ref[1] + ts_ref[...]) + b2_ref[...]


def _tc_out(u, ts, disb, b2):
    n, d = ts.shape
    r = 2000
    g = n // r
    return pl.pallas_call(
        _c3_body,
        grid=(g,),
        in_specs=[
            pl.BlockSpec((NC, r, d), lambda i: (0, i, 0)),
            pl.BlockSpec((r, d), lambda i: (i, 0)),
            pl.BlockSpec((r, d), lambda i: (i, 0)),
            pl.BlockSpec((1, d), lambda i: (0, 0)),
        ],
        out_specs=pl.BlockSpec((r, d), lambda i: (i, 0)),
        out_shape=jax.ShapeDtypeStruct((n, d), jnp.float32),
    )(u, ts, disb, b2)


def kernel(x, edge_index, W1, b1, W2, b2):
    n, d = x.shape
    e = edge_index.shape[1]

    tch = _round_up(_round_up(e, NS) // NS, 16 * CH) // CH
    nch0 = (tch // 4) // 4 * 4
    e_pad = NS * tch * CH
    n_chunk_w = tch // 2
    n_deg = _round_up(n + 1, NS * CH)
    n_acc = _round_up(n + 1, NS * CH)

    src = edge_index[0].astype(jnp.int32)
    dst = edge_index[1].astype(jnp.int32)
    pad = e_pad - e
    src_p = jnp.concatenate([src, jnp.zeros((pad,), jnp.int32)])
    dst_p = jnp.concatenate([dst, jnp.full((pad,), n, jnp.int32)])
    dst3 = dst_p.reshape(NW, n_chunk_w, CH)
    idx4 = jnp.stack([src_p.reshape(NS * tch, CH),
                      dst_p.reshape(NS * tch, CH)], axis=1)

    degp = _sc_degree(dst3, n_deg, n_chunk_w)
    degt = degp.T[:n]

    xs, disb = _tc_scale(x, degt)
    z = _sc_aggregate(xs, idx4, n, n_acc, tch, nch0)
    ts = _tc_layer(z, xs, disb, W1, b1.reshape(1, -1), W2)
    u = _sc_aggregate(ts, idx4, n, n_acc, tch, nch0)
    out = _tc_out(u, ts, disb, b2.reshape(1, -1))
    return out


# speedup vs baseline: 1.1252x; 1.1252x over previous
import functools

import jax
import jax.numpy as jnp
from jax import lax
from jax.experimental import pallas as pl
from jax.experimental.pallas import tpu as pltpu
from jax.experimental.pallas import tpu_sc as plsc

NC = 2
NS = 16
NW = NC * NS
CH = 128


def _round_up(a, b):
    return (a + b - 1) // b * b


def _sc_degree(dst3, n_deg, n_chunk):
    mesh = plsc.VectorSubcoreMesh(core_axis_name="c", subcore_axis_name="s")
    zt = n_deg // NS
    wave = 8

    @functools.partial(
        pl.kernel,
        out_type=jax.ShapeDtypeStruct((NC * n_deg,), jnp.float32),
        mesh=mesh,
        scratch_types=[
            pltpu.VMEM((n_chunk, CH), jnp.int32),
            pltpu.VMEM((CH,), jnp.float32),
            pltpu.VMEM((zt,), jnp.float32),
            pltpu.VMEM_SHARED((n_deg,), jnp.float32),
            pltpu.SemaphoreType.DMA,
        ],
    )
    def k(dst_h, out_h, dst_v, ones_v, stage_v, deg_sh, dsem):
        c = lax.axis_index("c")
        s = lax.axis_index("s")
        wid = s * NC + c
        pltpu.sync_copy(dst_h.at[wid], dst_v)
        for j in range(CH // 16):
            ones_v[pl.ds(16 * j, 16)] = jnp.ones((16,), jnp.float32)

        def zbody(i, carry):
            stage_v[pl.ds(i * 16, 16)] = jnp.zeros((16,), jnp.float32)
            return carry

        lax.fori_loop(0, zt // 16, zbody, 0)
        pltpu.sync_copy(stage_v, deg_sh.at[pl.ds(s * zt, zt)])
        plsc.subcore_barrier()

        def body(t, carry):
            j0 = t * wave
            for k_ in range(wave):
                pltpu.async_copy(ones_v, deg_sh.at[dst_v.at[j0 + k_]], dsem,
                                 add=True)
            for k_ in range(wave):
                pltpu.make_async_copy(ones_v, deg_sh.at[dst_v.at[0]],
                                      dsem).wait()
            return carry

        lax.fori_loop(0, n_chunk // wave, body, 0)
        plsc.subcore_barrier()
        pltpu.sync_copy(deg_sh.at[pl.ds(s * zt, zt)], stage_v)
        pltpu.sync_copy(stage_v, out_h.at[pl.ds(c * n_deg + s * zt, zt)])

    return k(dst3).reshape(NC, n_deg)


def _sc_aggregate(table, idx4, n, n_acc, tch, nch0):
    d = table.shape[1]
    mesh = plsc.VectorSubcoreMesh(core_axis_name="c", subcore_axis_name="s")
    zt = n_acc // NS
    ni = 4
    nr = 2
    assert nch0 % 4 == 0 and tch % 4 == 0
    assert min(nch0, tch - nch0) >= 12

    @functools.partial(
        pl.kernel,
        out_type=jax.ShapeDtypeStruct((NC, n_acc, d), jnp.float32),
        mesh=mesh,
        scratch_types=[
            pltpu.VMEM((ni, 2, CH), jnp.int32),
            pltpu.VMEM((nr, CH, d), jnp.float32),
            pltpu.VMEM_SHARED((n_acc, d), jnp.float32),
        ]
        + [pltpu.SemaphoreType.DMA] * (ni + 2 * nr),
    )
    def k(tab_h, idx_h, out_h, idx_v, rows_v, acc_sh, *sems):
        isems = sems[:ni]
        gsems = sems[ni:ni + nr]
        ssems = sems[ni + nr:]
        c = lax.axis_index("c")
        s = lax.axis_index("s")
        base = s * tch + jnp.where(c == 0, 0, nch0)
        ll = jnp.where(c == 0, nch0, tch - nch0)

        def istart(j, sl):
            pltpu.async_copy(idx_h.at[base + j], idx_v.at[sl], isems[sl])

        def iwait(sl):
            pltpu.make_async_copy(idx_h.at[0], idx_v.at[sl],
                                  isems[sl]).wait()

        def gstart(sl, rb):
            pltpu.async_copy(tab_h.at[idx_v.at[sl, 0]], rows_v.at[rb],
                             gsems[rb])

        def gwait(rb):
            pltpu.make_async_copy(tab_h.at[idx_v.at[0, 0]], rows_v.at[rb],
                                  gsems[rb]).wait()

        def sstart(sl, rb):
            pltpu.async_copy(rows_v.at[rb], acc_sh.at[idx_v.at[sl, 1]],
                             ssems[rb], add=True)

        def swait(rb):
            pltpu.make_async_copy(rows_v.at[rb], acc_sh.at[idx_v.at[0, 1]],
                                  ssems[rb]).wait()

        def zrow(i, carry):
            for j in range(d // 16):
                rows_v[0, i, pl.ds(j * 16, 16)] = jnp.zeros((16,), jnp.float32)
            return carry

        lax.fori_loop(0, CH, zrow, 0)

        def zcopy(i, carry):
            pltpu.sync_copy(rows_v.at[0], acc_sh.at[pl.ds(s * zt + i * CH, CH)])
            return carry

        lax.fori_loop(0, zt // CH, zcopy, 0)
        plsc.subcore_barrier()

        def step(j, sl, rb, do_ist):
            iwait((sl + 1) % ni)
            swait(1 - rb)
            gstart((sl + 1) % ni, 1 - rb)
            gwait(rb)
            sstart(sl, rb)
            if do_ist:
                istart(j + 3, (sl + 3) % ni)

        istart(0, 0)
        istart(1, 1)
        istart(2, 2)
        iwait(0)
        gstart(0, 0)
        iwait(1)
        gstart(1, 1)
        gwait(0)
        sstart(0, 0)
        istart(3, 3)
        for j in (1, 2, 3):
            step(j, j % ni, j % nr, True)

        def mbody(t, carry):
            j0 = 4 + t * 4
            for k_ in range(4):
                step(j0 + k_, k_ % ni, k_ % nr, True)
            return carry

        lax.fori_loop(0, (ll - 8) // 4, mbody, 0)
        for k_ in range(3):
            step(ll - 4 + k_, k_, k_ % nr, k_ == 0)
        swait(0)
        gwait(1)
        sstart(3, 1)
        swait(1)
        plsc.subcore_barrier()

        def ocopy(i, carry):
            r0 = s * zt + i * CH
            pltpu.sync_copy(acc_sh.at[pl.ds(r0, CH)], rows_v.at[0])
            pltpu.sync_copy(rows_v.at[0], out_h.at[c, pl.ds(r0, CH)])
            return carry

        lax.fori_loop(0, zt // CH, ocopy, 0)

    return k(table, idx4)


def _c1_body(x_ref, dt_ref, xs_ref, disb_ref):
    dt = dt_ref[...]
    deg = dt[:, 0:1] + dt[:, 1:2] + 1.0
    dis = lax.rsqrt(deg)
    disb = jnp.broadcast_to(dis, x_ref.shape)
    disb_ref[...] = disb
    xs_ref[...] = x_ref[...] * disb


def _tc_scale(x, degt):
    n, d = x.shape
    r = 2000
    g = n // r
    return pl.pallas_call(
        _c1_body,
        grid=(g,),
        in_specs=[
            pl.BlockSpec((r, d), lambda i: (i, 0)),
            pl.BlockSpec((r, NC), lambda i: (i, 0)),
        ],
        out_specs=[
            pl.BlockSpec((r, d), lambda i: (i, 0)),
            pl.BlockSpec((r, d), lambda i: (i, 0)),
        ],
        out_shape=[
            jax.ShapeDtypeStruct((n, d), jnp.float32),
            jax.ShapeDtypeStruct((n, d), jnp.float32),
        ],
    )(x, degt)


def _c2_body(z_ref, xs_ref, disb_ref, w1_ref, b1_ref, w2_ref, o_ref):
    disb = disb_ref[...]
    a = disb * (z_ref[0] + z_ref[1] + xs_ref[...])
    h = jnp.dot(a, w1_ref[...], preferred_element_type=jnp.float32,
                precision=lax.Precision.HIGHEST)
    h = jnp.maximum(h + b1_ref[...], 0.0)
    t = jnp.dot(h, w2_ref[...], preferred_element_type=jnp.float32,
                precision=lax.Precision.HIGHEST)
    o_ref[...] = disb * t


def _tc_layer(z, xs, disb, W1, b1, W2):
    n, d = xs.shape
    h = W1.shape[1]
    r = 2000
    g = n // r
    return pl.pallas_call(
        _c2_body,
        grid=(g,),
        in_specs=[
            pl.BlockSpec((NC, r, d), lambda i: (0, i, 0)),
            pl.BlockSpec((r, d), lambda i: (i, 0)),
            pl.BlockSpec((r, d), lambda i: (i, 0)),
            pl.BlockSpec((d, h), lambda i: (0, 0)),
            pl.BlockSpec((1, h), lambda i: (0, 0)),
            pl.BlockSpec((h, d), lambda i: (0, 0)),
        ],
        out_specs=pl.BlockSpec((r, d), lambda i: (i, 0)),
        out_shape=jax.ShapeDtypeStruct((n, d), jnp.float32),
    )(z, xs, disb, W1, b1, W2)


def _c3_body(u_ref, ts_ref, disb_ref, b2_ref, o_ref):
    o_ref[...] = disb_ref[...] * (u_ref[0] + u_ref[1] + ts_ref[...]) + b2_ref[...]


def _tc_out(u, ts, disb, b2):
    n, d = ts.shape
    r = 2000
    g = n // r
    return pl.pallas_call(
        _c3_body,
        grid=(g,),
        in_specs=[
            pl.BlockSpec((NC, r, d), lambda i: (0, i, 0)),
            pl.BlockSpec((r, d), lambda i: (i, 0)),
            pl.BlockSpec((r, d), lambda i: (i, 0)),
            pl.BlockSpec((1, d), lambda i: (0, 0)),
        ],
        out_specs=pl.BlockSpec((r, d), lambda i: (i, 0)),
        out_shape=jax.ShapeDtypeStruct((n, d), jnp.float32),
    )(u, ts, disb, b2)


def kernel(x, edge_index, W1, b1, W2, b2):
    n, d = x.shape
    e = edge_index.shape[1]

    tch = _round_up(_round_up(e, NS) // NS, 16 * CH) // CH
    nch0 = (tch * 3 // 4) // 4 * 4
    e_pad = NS * tch * CH
    n_chunk_w = tch // 2
    n_deg = _round_up(n + 1, NS * CH)
    n_acc = _round_up(n + 1, NS * CH)

    src = edge_index[0].astype(jnp.int32)
    dst = edge_index[1].astype(jnp.int32)
    pad = e_pad - e
    src_p = jnp.concatenate([src, jnp.zeros((pad,), jnp.int32)])
    dst_p = jnp.concatenate([dst, jnp.full((pad,), n, jnp.int32)])
    dst3 = dst_p.reshape(NW, n_chunk_w, CH)
    idx4 = jnp.stack([src_p.reshape(NS * tch, CH),
                      dst_p.reshape(NS * tch, CH)], axis=1)

    degp = _sc_degree(dst3, n_deg, n_chunk_w)
    degt = degp.T[:n]

    xs, disb = _tc_scale(x, degt)
    z = _sc_aggregate(xs, idx4, n, n_acc, tch, nch0)
    ts = _tc_layer(z, xs, disb, W1, b1.reshape(1, -1), W2)
    u = _sc_aggregate(ts, idx4, n, n_acc, tch, nch0)
    out = _tc_out(u, ts, disb, b2.reshape(1, -1))
    return out


# speedup vs baseline: 1.1576x; 1.0287x over previous
import functools

import jax
import jax.numpy as jnp
from jax import lax
from jax.experimental import pallas as pl
from jax.experimental.pallas import tpu as pltpu
from jax.experimental.pallas import tpu_sc as plsc

NC = 2
NS = 16
NW = NC * NS
CH = 128


def _round_up(a, b):
    return (a + b - 1) // b * b


def _sc_degree(dst3, n_deg, n_chunk):
    mesh = plsc.VectorSubcoreMesh(core_axis_name="c", subcore_axis_name="s")
    zt = n_deg // NS
    wave = 8

    @functools.partial(
        pl.kernel,
        out_type=jax.ShapeDtypeStruct((NC * n_deg,), jnp.float32),
        mesh=mesh,
        scratch_types=[
            pltpu.VMEM((n_chunk, CH), jnp.int32),
            pltpu.VMEM((CH,), jnp.float32),
            pltpu.VMEM((zt,), jnp.float32),
            pltpu.VMEM_SHARED((n_deg,), jnp.float32),
            pltpu.SemaphoreType.DMA,
        ],
    )
    def k(dst_h, out_h, dst_v, ones_v, stage_v, deg_sh, dsem):
        c = lax.axis_index("c")
        s = lax.axis_index("s")
        wid = s * NC + c
        pltpu.sync_copy(dst_h.at[wid], dst_v)
        for j in range(CH // 16):
            ones_v[pl.ds(16 * j, 16)] = jnp.ones((16,), jnp.float32)

        def zbody(i, carry):
            stage_v[pl.ds(i * 16, 16)] = jnp.zeros((16,), jnp.float32)
            return carry

        lax.fori_loop(0, zt // 16, zbody, 0)
        pltpu.sync_copy(stage_v, deg_sh.at[pl.ds(s * zt, zt)])
        plsc.subcore_barrier()

        def body(t, carry):
            j0 = t * wave
            for k_ in range(wave):
                pltpu.async_copy(ones_v, deg_sh.at[dst_v.at[j0 + k_]], dsem,
                                 add=True)
            for k_ in range(wave):
                pltpu.make_async_copy(ones_v, deg_sh.at[dst_v.at[0]],
                                      dsem).wait()
            return carry

        lax.fori_loop(0, n_chunk // wave, body, 0)
        plsc.subcore_barrier()
        pltpu.sync_copy(deg_sh.at[pl.ds(s * zt, zt)], stage_v)
        pltpu.sync_copy(stage_v, out_h.at[pl.ds(c * n_deg + s * zt, zt)])

    return k(dst3).reshape(NC, n_deg)


def _sc_aggregate(table, idx4, n, n_acc, tch, nch0):
    d = table.shape[1]
    dt = table.dtype
    mesh = plsc.VectorSubcoreMesh(core_axis_name="c", subcore_axis_name="s")
    zt = n_acc // NS
    ni = 4
    nr = 2
    vl = 32 if dt == jnp.bfloat16 else 16
    assert nch0 % 4 == 0 and tch % 4 == 0
    assert min(nch0, tch - nch0) >= 12

    @functools.partial(
        pl.kernel,
        out_type=jax.ShapeDtypeStruct((NC, n_acc, d), dt),
        mesh=mesh,
        scratch_types=[
            pltpu.VMEM((ni, 2, CH), jnp.int32),
            pltpu.VMEM((nr, CH, d), dt),
            pltpu.VMEM_SHARED((n_acc, d), dt),
        ]
        + [pltpu.SemaphoreType.DMA] * (ni + 2 * nr),
    )
    def k(tab_h, idx_h, out_h, idx_v, rows_v, acc_sh, *sems):
        isems = sems[:ni]
        gsems = sems[ni:ni + nr]
        ssems = sems[ni + nr:]
        c = lax.axis_index("c")
        s = lax.axis_index("s")
        base = s * tch + jnp.where(c == 0, 0, nch0)
        ll = jnp.where(c == 0, nch0, tch - nch0)

        def istart(j, sl):
            pltpu.async_copy(idx_h.at[base + j], idx_v.at[sl], isems[sl])

        def iwait(sl):
            pltpu.make_async_copy(idx_h.at[0], idx_v.at[sl],
                                  isems[sl]).wait()

        def gstart(sl, rb):
            pltpu.async_copy(tab_h.at[idx_v.at[sl, 0]], rows_v.at[rb],
                             gsems[rb])

        def gwait(rb):
            pltpu.make_async_copy(tab_h.at[idx_v.at[0, 0]], rows_v.at[rb],
                                  gsems[rb]).wait()

        def sstart(sl, rb):
            pltpu.async_copy(rows_v.at[rb], acc_sh.at[idx_v.at[sl, 1]],
                             ssems[rb], add=True)

        def swait(rb):
            pltpu.make_async_copy(rows_v.at[rb], acc_sh.at[idx_v.at[0, 1]],
                                  ssems[rb]).wait()

        if dt == jnp.bfloat16:
            def zrow(i, carry):
                for j in range(d // 16):
                    rows_v[0, pl.ds(2 * i, 2), pl.ds(j * 16, 16)] = (
                        jnp.zeros((2, 16), dt))
                return carry

            lax.fori_loop(0, CH // 2, zrow, 0)
        else:
            def zrow(i, carry):
                for j in range(d // vl):
                    rows_v[0, i, pl.ds(j * vl, vl)] = jnp.zeros((vl,), dt)
                return carry

            lax.fori_loop(0, CH, zrow, 0)

        def zcopy(i, carry):
            pltpu.sync_copy(rows_v.at[0], acc_sh.at[pl.ds(s * zt + i * CH, CH)])
            return carry

        lax.fori_loop(0, zt // CH, zcopy, 0)
        plsc.subcore_barrier()

        def step(j, sl, rb, do_ist):
            iwait((sl + 1) % ni)
            swait(1 - rb)
            gstart((sl + 1) % ni, 1 - rb)
            gwait(rb)
            sstart(sl, rb)
            if do_ist:
                istart(j + 3, (sl + 3) % ni)

        istart(0, 0)
        istart(1, 1)
        istart(2, 2)
        iwait(0)
        gstart(0, 0)
        iwait(1)
        gstart(1, 1)
        gwait(0)
        sstart(0, 0)
        istart(3, 3)
        for j in (1, 2, 3):
            step(j, j % ni, j % nr, True)

        def mbody(t, carry):
            j0 = 4 + t * 4
            for k_ in range(4):
                step(j0 + k_, k_ % ni, k_ % nr, True)
            return carry

        lax.fori_loop(0, (ll - 8) // 4, mbody, 0)
        for k_ in range(3):
            step(ll - 4 + k_, k_, k_ % nr, k_ == 0)
        swait(0)
        gwait(1)
        sstart(3, 1)
        swait(1)
        plsc.subcore_barrier()

        def ocopy(i, carry):
            r0 = s * zt + i * CH
            pltpu.sync_copy(acc_sh.at[pl.ds(r0, CH)], rows_v.at[0])
            pltpu.sync_copy(rows_v.at[0], out_h.at[c, pl.ds(r0, CH)])
            return carry

        lax.fori_loop(0, zt // CH, ocopy, 0)

    return k(table, idx4)


def _c1_body(x_ref, dt_ref, xs_ref, disb_ref):
    dt = dt_ref[...]
    deg = dt[:, 0:1] + dt[:, 1:2] + 1.0
    dis = lax.rsqrt(deg)
    disb = jnp.broadcast_to(dis, x_ref.shape)
    disb_ref[...] = disb
    xs_ref[...] = x_ref[...] * disb


def _tc_scale(x, degt):
    n, d = x.shape
    r = 2000
    g = n // r
    return pl.pallas_call(
        _c1_body,
        grid=(g,),
        in_specs=[
            pl.BlockSpec((r, d), lambda i: (i, 0)),
            pl.BlockSpec((r, NC), lambda i: (i, 0)),
        ],
        out_specs=[
            pl.BlockSpec((r, d), lambda i: (i, 0)),
            pl.BlockSpec((r, d), lambda i: (i, 0)),
        ],
        out_shape=[
            jax.ShapeDtypeStruct((n, d), jnp.float32),
            jax.ShapeDtypeStruct((n, d), jnp.float32),
        ],
    )(x, degt)


def _c2_body(z_ref, xs_ref, disb_ref, w1_ref, b1_ref, w2_ref, o_ref):
    disb = disb_ref[...]
    zsum = (z_ref[0].astype(jnp.float32) + z_ref[1].astype(jnp.float32))
    a = disb * (zsum + xs_ref[...])
    h = jnp.dot(a, w1_ref[...], preferred_element_type=jnp.float32)
    h = jnp.maximum(h + b1_ref[...], 0.0)
    t = jnp.dot(h, w2_ref[...], preferred_element_type=jnp.float32)
    o_ref[...] = disb * t


def _tc_layer(z, xs, disb, W1, b1, W2):
    n, d = xs.shape
    h = W1.shape[1]
    r = 2000
    g = n // r
    return pl.pallas_call(
        _c2_body,
        grid=(g,),
        in_specs=[
            pl.BlockSpec((NC, r, d), lambda i: (0, i, 0)),
            pl.BlockSpec((r, d), lambda i: (i, 0)),
            pl.BlockSpec((r, d), lambda i: (i, 0)),
            pl.BlockSpec((d, h), lambda i: (0, 0)),
            pl.BlockSpec((1, h), lambda i: (0, 0)),
            pl.BlockSpec((h, d), lambda i: (0, 0)),
        ],
        out_specs=pl.BlockSpec((r, d), lambda i: (i, 0)),
        out_shape=jax.ShapeDtypeStruct((n, d), jnp.float32),
    )(z, xs, disb, W1, b1, W2)


def _c3_body(u_ref, ts_ref, disb_ref, b2_ref, o_ref):
    usum = (u_ref[0].astype(jnp.float32) + u_ref[1].astype(jnp.float32))
    o_ref[...] = disb_ref[...] * (usum + ts_ref[...]) + b2_ref[...]


def _tc_out(u, ts, disb, b2):
    n, d = ts.shape
    r = 2000
    g = n // r
    return pl.pallas_call(
        _c3_body,
        grid=(g,),
        in_specs=[
            pl.BlockSpec((NC, r, d), lambda i: (0, i, 0)),
            pl.BlockSpec((r, d), lambda i: (i, 0)),
            pl.BlockSpec((r, d), lambda i: (i, 0)),
            pl.BlockSpec((1, d), lambda i: (0, 0)),
        ],
        out_specs=pl.BlockSpec((r, d), lambda i: (i, 0)),
        out_shape=jax.ShapeDtypeStruct((n, d), jnp.float32),
    )(u, ts, disb, b2)


def kernel(x, edge_index, W1, b1, W2, b2):
    n, d = x.shape
    e = edge_index.shape[1]

    tch = _round_up(_round_up(e, NS) // NS, 16 * CH) // CH
    nch0 = (tch * 31 // 40) // 4 * 4
    e_pad = NS * tch * CH
    n_chunk_w = tch // 2
    n_deg = _round_up(n + 1, NS * CH)
    n_acc = _round_up(n + 1, NS * CH)

    src = edge_index[0].astype(jnp.int32)
    dst = edge_index[1].astype(jnp.int32)
    pad = e_pad - e
    src_p = jnp.concatenate([src, jnp.zeros((pad,), jnp.int32)])
    dst_p = jnp.concatenate([dst, jnp.full((pad,), n, jnp.int32)])
    dst3 = dst_p.reshape(NW, n_chunk_w, CH)
    idx4 = jnp.stack([src_p.reshape(NS * tch, CH),
                      dst_p.reshape(NS * tch, CH)], axis=1)

    degp = _sc_degree(dst3, n_deg, n_chunk_w)
    degt = degp.T[:n]

    xs, disb = _tc_scale(x, degt)
    z = _sc_aggregate(xs, idx4, n, n_acc, tch, nch0)
    ts = _tc_layer(z, xs, disb, W1, b1.reshape(1, -1), W2)
    u = _sc_aggregate(ts, idx4, n, n_acc, tch, nch0)
    out = _tc_out(u, ts, disb, b2.reshape(1, -1))
    return out
